# baseline (speedup 1.0000x reference)
import functools

import jax
import jax.numpy as jnp
from jax import lax
from jax.experimental import pallas as pl
from jax.experimental.pallas import tpu as pltpu
from jax.experimental.pallas import tpu_sc as plsc

_NC = 2
_NS = 16
_NW = _NC * _NS
_SUBCHUNKS = 4


def _sc_copy_body(rows_per_worker, cols, in_hbm, out_hbm, buf, in_sems, out_sems):
    wid = lax.axis_index("s") * _NC + lax.axis_index("c")
    sub = rows_per_worker // _SUBCHUNKS
    base = wid * rows_per_worker

    def cin(j):
        return pltpu.make_async_copy(
            in_hbm.at[pl.ds(base + j * sub, sub)], buf.at[j], in_sems.at[j])

    def cout(j):
        return pltpu.make_async_copy(
            buf.at[j], out_hbm.at[pl.ds(base + j * sub, sub)], out_sems.at[j])

    for j in range(_SUBCHUNKS):
        cin(j).start()
    for j in range(_SUBCHUNKS):
        cin(j).wait()
        cout(j).start()
    for j in range(_SUBCHUNKS):
        cout(j).wait()


def kernel(inputs, embedding_table):
    del embedding_table
    rows, cols = inputs.shape
    rows_per_worker = rows // _NW
    sub = rows_per_worker // _SUBCHUNKS
    mesh = plsc.VectorSubcoreMesh(core_axis_name="c", subcore_axis_name="s")
    sc_copy = pl.kernel(
        functools.partial(_sc_copy_body, rows_per_worker, cols),
        out_type=jax.ShapeDtypeStruct(inputs.shape, inputs.dtype),
        mesh=mesh,
        scratch_types=[
            pltpu.VMEM((_SUBCHUNKS, sub, cols), inputs.dtype),
            pltpu.SemaphoreType.DMA((_SUBCHUNKS,)),
            pltpu.SemaphoreType.DMA((_SUBCHUNKS,)),
        ],
    )
    return sc_copy(inputs)


# speedup vs baseline: 1.3885x; 1.3885x over previous
import jax
import jax.numpy as jnp
from jax.experimental import pallas as pl
from jax.experimental.pallas import tpu as pltpu

_NUM_CHUNKS = 32


def _copy_kernel(in_hbm, out_hbm, buf, in_sems, out_sems):
    rows, _ = in_hbm.shape
    chunk = rows // _NUM_CHUNKS

    def copy_in(i):
        return pltpu.make_async_copy(
            in_hbm.at[pl.ds(i * chunk, chunk)], buf.at[i], in_sems.at[i])

    def copy_out(i):
        return pltpu.make_async_copy(
            buf.at[i], out_hbm.at[pl.ds(i * chunk, chunk)], out_sems.at[i])

    for i in range(_NUM_CHUNKS):
        copy_in(i).start()
    for i in range(_NUM_CHUNKS):
        copy_in(i).wait()
        copy_out(i).start()
    for i in range(_NUM_CHUNKS):
        copy_out(i).wait()


def kernel(inputs, embedding_table):
    del embedding_table
    rows, cols = inputs.shape
    chunk = rows // _NUM_CHUNKS
    return pl.pallas_call(
        _copy_kernel,
        out_shape=jax.ShapeDtypeStruct(inputs.shape, inputs.dtype),
        in_specs=[pl.BlockSpec(memory_space=pl.ANY)],
        out_specs=pl.BlockSpec(memory_space=pl.ANY),
        scratch_shapes=[
            pltpu.VMEM((_NUM_CHUNKS, chunk, cols), inputs.dtype),
            pltpu.SemaphoreType.DMA((_NUM_CHUNKS,)),
            pltpu.SemaphoreType.DMA((_NUM_CHUNKS,)),
        ],
    )(inputs)
